# baseline (speedup 1.0000x reference)
import jax
import jax.numpy as jnp
from jax.experimental import pallas as pl
from jax.experimental.pallas import tpu as pltpu

N_J = 30
N_M = 20
B = 8
NPG = N_J * N_M
N = B * NPG
HID = 64
BNJ = B * N_J
BLK = 80
K = N // BLK
PB = 320
PC = N // PB

_bf = lambda x: x.astype(jnp.bfloat16)


def _bn(t, g, b, eps=1e-5):
    mu = jnp.mean(t, axis=0, keepdims=True)
    var = jnp.mean((t - mu) ** 2, axis=0, keepdims=True)
    return g * (t - mu) / jnp.sqrt(var + eps) + b


def _body(adj_blk_ref, feat_ref, gp_ref,
          w10_ref, b10_ref, w20_ref, b20_ref, mg0_ref, mb0_ref, og0_ref, ob0_ref,
          w11_ref, b11_ref, w21_ref, b21_ref, mg1_ref, mb1_ref, og1_ref, ob1_ref,
          h2_ref, pooled_ref,
          adj_bf_ref, t_ref, s1_ref, s2_ref):
    i = pl.program_id(0)

    @pl.when(i < K)
    def _stream():
        blk = _bf(adj_blk_ref[...])
        adj_bf_ref[pl.ds(i * BLK, BLK), :] = blk
        p0 = jnp.dot(blk, feat_ref[...], preferred_element_type=jnp.float32)
        t_ref[pl.ds(i * BLK, BLK), :] = jnp.dot(
            _bf(p0), w10_ref[...], preferred_element_type=jnp.float32)

    @pl.when(i == K)
    def _tail0():
        t0 = t_ref[...] + b10_ref[...]
        s1_ref[...] = _bf(jax.nn.relu(_bn(t0, mg0_ref[...], mb0_ref[...])))

    @pl.when(i == K + 1)
    def _tail1():
        rep = jnp.dot(s1_ref[...], w20_ref[...],
                      preferred_element_type=jnp.float32) + b20_ref[...]
        s1_ref[...] = _bf(jax.nn.relu(_bn(rep, og0_ref[...], ob0_ref[...])))

    @pl.when(jnp.logical_and(i >= K + 2, i < K + 2 + PC))
    def _tail2():
        c = i - (K + 2)
        s2_ref[pl.ds(c * PB, PB), :] = jnp.dot(
            adj_bf_ref[pl.ds(c * PB, PB), :], s1_ref[...],
            preferred_element_type=jnp.float32)

    @pl.when(i == K + 2 + PC)
    def _tail3():
        t1 = jnp.dot(_bf(s2_ref[...]), w11_ref[...],
                     preferred_element_type=jnp.float32) + b11_ref[...]
        s1_ref[...] = _bf(jax.nn.relu(_bn(t1, mg1_ref[...], mb1_ref[...])))

    @pl.when(i == K + 3 + PC)
    def _tail4():
        rep1 = jnp.dot(s1_ref[...], w21_ref[...],
                       preferred_element_type=jnp.float32) + b21_ref[...]
        h2 = jax.nn.relu(_bn(rep1, og1_ref[...], ob1_ref[...]))
        h2_ref[...] = h2
        pooled_ref[...] = jnp.dot(gp_ref[...], _bf(h2),
                                  preferred_element_type=jnp.float32)


def _gather_body(cand_ref, h2_ref, hp_ref, concat_ref):
    col = jax.lax.broadcasted_iota(jnp.int32, (BNJ, N), 1)
    onehot = (cand_ref[...] == col).astype(jnp.bfloat16)
    concat_ref[:, 0:HID] = jnp.dot(onehot, _bf(h2_ref[...]),
                                   preferred_element_type=jnp.float32)
    hp = hp_ref[...]
    for b in range(B):
        concat_ref[pl.ds(b * N_J, N_J), HID:2 * HID] = jnp.broadcast_to(
            hp[b:b + 1, :], (N_J, HID))


def kernel(adj, features, candidate, graph_pool,
           w1_0, b1_0, w2_0, b2_0, mlp_bn_g_0, mlp_bn_b_0, bn_g_0, bn_b_0,
           w1_1, b1_1, w2_1, b2_1, mlp_bn_g_1, mlp_bn_b_1, bn_g_1, bn_b_1):
    vec = lambda v: v.reshape(1, HID)
    full = lambda arr: pl.BlockSpec(arr.shape, lambda i: (0,) * arr.ndim)
    small = (_bf(graph_pool),
             _bf(w1_0), vec(b1_0), _bf(w2_0), vec(b2_0),
             vec(mlp_bn_g_0), vec(mlp_bn_b_0), vec(bn_g_0), vec(bn_b_0),
             _bf(w1_1), vec(b1_1), _bf(w2_1), vec(b2_1),
             vec(mlp_bn_g_1), vec(mlp_bn_b_1), vec(bn_g_1), vec(bn_b_1))
    h2, pooled = pl.pallas_call(
        _body,
        grid=(K + PC + 4,),
        in_specs=[
            pl.BlockSpec((BLK, N), lambda i: (jnp.minimum(i, K - 1), 0)),
            full(features),
        ] + [full(a) for a in small],
        out_specs=(
            pl.BlockSpec((N, HID), lambda i: (0, 0)),
            pl.BlockSpec((B, HID), lambda i: (0, 0)),
        ),
        out_shape=(
            jax.ShapeDtypeStruct((N, HID), jnp.float32),
            jax.ShapeDtypeStruct((B, HID), jnp.float32),
        ),
        scratch_shapes=[
            pltpu.VMEM((N, N), jnp.bfloat16),
            pltpu.VMEM((N, HID), jnp.float32),
            pltpu.VMEM((N, HID), jnp.bfloat16),
            pltpu.VMEM((N, HID), jnp.float32),
        ],
        compiler_params=pltpu.CompilerParams(
            vmem_limit_bytes=128 * 1024 * 1024,
        ),
    )(adj, _bf(features), *small)

    cand2 = (candidate.astype(jnp.int32)
             + jnp.arange(B, dtype=jnp.int32)[:, None] * NPG).reshape(BNJ, 1)
    concat2 = pl.pallas_call(
        _gather_body,
        out_shape=jax.ShapeDtypeStruct((BNJ, 2 * HID), jnp.float32),
    )(cand2, h2, pooled)
    return (concat2.reshape(B, N_J, 2 * HID), pooled)


# speedup vs baseline: 1.2422x; 1.2422x over previous
import jax
import jax.numpy as jnp
from jax.experimental import pallas as pl
from jax.experimental.pallas import tpu as pltpu

N_J = 30
N_M = 20
B = 8
NPG = N_J * N_M
N = B * NPG
HID = 64
BNJ = B * N_J
BLK = 160
K = N // BLK
PB = 480
PC = N // PB

_bf = lambda x: x.astype(jnp.bfloat16)


def _bn(t, g, b, eps=1e-5):
    mu = jnp.mean(t, axis=0, keepdims=True)
    var = jnp.mean((t - mu) ** 2, axis=0, keepdims=True)
    return g * (t - mu) / jnp.sqrt(var + eps) + b


def _body(adj_blk_ref, feat_ref, gp_ref,
          w10_ref, b10_ref, w20_ref, b20_ref, mg0_ref, mb0_ref, og0_ref, ob0_ref,
          w11_ref, b11_ref, w21_ref, b21_ref, mg1_ref, mb1_ref, og1_ref, ob1_ref,
          h2_ref, pooled_ref,
          adj_bf_ref, t_ref, s1_ref, s2_ref):
    i = pl.program_id(0)

    @pl.when(i < K)
    def _stream():
        blk = _bf(adj_blk_ref[...])
        adj_bf_ref[pl.ds(i * BLK, BLK), :] = blk
        p0 = jnp.dot(blk, feat_ref[...], preferred_element_type=jnp.float32)
        t_ref[pl.ds(i * BLK, BLK), :] = jnp.dot(
            _bf(p0), w10_ref[...], preferred_element_type=jnp.float32)

    @pl.when(i == K)
    def _tail0():
        t0 = t_ref[...] + b10_ref[...]
        s1_ref[...] = _bf(jax.nn.relu(_bn(t0, mg0_ref[...], mb0_ref[...])))

    @pl.when(i == K + 1)
    def _tail1():
        rep = jnp.dot(s1_ref[...], w20_ref[...],
                      preferred_element_type=jnp.float32) + b20_ref[...]
        s1_ref[...] = _bf(jax.nn.relu(_bn(rep, og0_ref[...], ob0_ref[...])))

    @pl.when(jnp.logical_and(i >= K + 2, i < K + 2 + PC))
    def _tail2():
        c = i - (K + 2)
        s2_ref[pl.ds(c * PB, PB), :] = jnp.dot(
            adj_bf_ref[pl.ds(c * PB, PB), :], s1_ref[...],
            preferred_element_type=jnp.float32)

    @pl.when(i == K + 2 + PC)
    def _tail3():
        t1 = jnp.dot(_bf(s2_ref[...]), w11_ref[...],
                     preferred_element_type=jnp.float32) + b11_ref[...]
        s1_ref[...] = _bf(jax.nn.relu(_bn(t1, mg1_ref[...], mb1_ref[...])))

    @pl.when(i == K + 3 + PC)
    def _tail4():
        rep1 = jnp.dot(s1_ref[...], w21_ref[...],
                       preferred_element_type=jnp.float32) + b21_ref[...]
        h2 = jax.nn.relu(_bn(rep1, og1_ref[...], ob1_ref[...]))
        h2_ref[...] = h2
        pooled_ref[...] = jnp.dot(gp_ref[...], _bf(h2),
                                  preferred_element_type=jnp.float32)


def _gather_body(cand_ref, h2_ref, hp_ref, concat_ref):
    col = jax.lax.broadcasted_iota(jnp.int32, (BNJ, N), 1)
    onehot = (cand_ref[...] == col).astype(jnp.bfloat16)
    concat_ref[:, 0:HID] = jnp.dot(onehot, _bf(h2_ref[...]),
                                   preferred_element_type=jnp.float32)
    hp = hp_ref[...]
    for b in range(B):
        concat_ref[pl.ds(b * N_J, N_J), HID:2 * HID] = jnp.broadcast_to(
            hp[b:b + 1, :], (N_J, HID))


def kernel(adj, features, candidate, graph_pool,
           w1_0, b1_0, w2_0, b2_0, mlp_bn_g_0, mlp_bn_b_0, bn_g_0, bn_b_0,
           w1_1, b1_1, w2_1, b2_1, mlp_bn_g_1, mlp_bn_b_1, bn_g_1, bn_b_1):
    vec = lambda v: v.reshape(1, HID)
    full = lambda arr: pl.BlockSpec(arr.shape, lambda i: (0,) * arr.ndim)
    small = (_bf(graph_pool),
             _bf(w1_0), vec(b1_0), _bf(w2_0), vec(b2_0),
             vec(mlp_bn_g_0), vec(mlp_bn_b_0), vec(bn_g_0), vec(bn_b_0),
             _bf(w1_1), vec(b1_1), _bf(w2_1), vec(b2_1),
             vec(mlp_bn_g_1), vec(mlp_bn_b_1), vec(bn_g_1), vec(bn_b_1))
    h2, pooled = pl.pallas_call(
        _body,
        grid=(K + PC + 4,),
        in_specs=[
            pl.BlockSpec((BLK, N), lambda i: (jnp.minimum(i, K - 1), 0)),
            full(features),
        ] + [full(a) for a in small],
        out_specs=(
            pl.BlockSpec((N, HID), lambda i: (0, 0)),
            pl.BlockSpec((B, HID), lambda i: (0, 0)),
        ),
        out_shape=(
            jax.ShapeDtypeStruct((N, HID), jnp.float32),
            jax.ShapeDtypeStruct((B, HID), jnp.float32),
        ),
        scratch_shapes=[
            pltpu.VMEM((N, N), jnp.bfloat16),
            pltpu.VMEM((N, HID), jnp.float32),
            pltpu.VMEM((N, HID), jnp.bfloat16),
            pltpu.VMEM((N, HID), jnp.float32),
        ],
        compiler_params=pltpu.CompilerParams(
            vmem_limit_bytes=128 * 1024 * 1024,
        ),
    )(adj, _bf(features), *small)

    cand2 = (candidate.astype(jnp.int32)
             + jnp.arange(B, dtype=jnp.int32)[:, None] * NPG).reshape(BNJ, 1)
    concat2 = pl.pallas_call(
        _gather_body,
        out_shape=jax.ShapeDtypeStruct((BNJ, 2 * HID), jnp.float32),
    )(cand2, h2, pooled)
    return (concat2.reshape(B, N_J, 2 * HID), pooled)


# speedup vs baseline: 1.3482x; 1.0853x over previous
import jax
import jax.numpy as jnp
from jax.experimental import pallas as pl
from jax.experimental.pallas import tpu as pltpu

N_J = 30
N_M = 20
B = 8
NPG = N_J * N_M
N = B * NPG
HID = 64
BNJ = B * N_J
BLK = 240
K = N // BLK
PB = 480
PC = N // PB

_bf = lambda x: x.astype(jnp.bfloat16)


def _bn(t, g, b, eps=1e-5):
    mu = jnp.mean(t, axis=0, keepdims=True)
    var = jnp.mean((t - mu) ** 2, axis=0, keepdims=True)
    return g * (t - mu) / jnp.sqrt(var + eps) + b


def _body(adj_blk_ref, feat_ref, gp_ref,
          w10_ref, b10_ref, w20_ref, b20_ref, mg0_ref, mb0_ref, og0_ref, ob0_ref,
          w11_ref, b11_ref, w21_ref, b21_ref, mg1_ref, mb1_ref, og1_ref, ob1_ref,
          h2_ref, pooled_ref,
          adj_bf_ref, t_ref, s1_ref):
    i = pl.program_id(0)

    @pl.when(i < K)
    def _stream():
        blk = _bf(adj_blk_ref[...])
        adj_bf_ref[pl.ds(i * BLK, BLK), :] = blk
        p0 = jnp.dot(blk, feat_ref[...], preferred_element_type=jnp.float32)
        t_ref[pl.ds(i * BLK, BLK), :] = jnp.dot(
            _bf(p0), w10_ref[...], preferred_element_type=jnp.float32)

    @pl.when(i == K)
    def _tail0():
        t0 = t_ref[...] + b10_ref[...]
        s1_ref[...] = _bf(jax.nn.relu(_bn(t0, mg0_ref[...], mb0_ref[...])))

    @pl.when(i == K + 1)
    def _tail1():
        rep = jnp.dot(s1_ref[...], w20_ref[...],
                      preferred_element_type=jnp.float32) + b20_ref[...]
        s1_ref[...] = _bf(jax.nn.relu(_bn(rep, og0_ref[...], ob0_ref[...])))

    @pl.when(jnp.logical_and(i >= K + 2, i < K + 2 + PC))
    def _tail2():
        c = i - (K + 2)
        t_ref[pl.ds(c * PB, PB), :] = jnp.dot(
            adj_bf_ref[pl.ds(c * PB, PB), :], s1_ref[...],
            preferred_element_type=jnp.float32)

    @pl.when(i == K + 2 + PC)
    def _tail3():
        t1 = jnp.dot(_bf(t_ref[...]), w11_ref[...],
                     preferred_element_type=jnp.float32) + b11_ref[...]
        s1_ref[...] = _bf(jax.nn.relu(_bn(t1, mg1_ref[...], mb1_ref[...])))

    @pl.when(i == K + 3 + PC)
    def _tail4():
        rep1 = jnp.dot(s1_ref[...], w21_ref[...],
                       preferred_element_type=jnp.float32) + b21_ref[...]
        h2 = jax.nn.relu(_bn(rep1, og1_ref[...], ob1_ref[...]))
        h2_ref[...] = h2
        pooled_ref[...] = jnp.dot(gp_ref[...], _bf(h2),
                                  preferred_element_type=jnp.float32)


def _gather_body(cand_ref, h2_ref, hp_ref, concat_ref):
    col = jax.lax.broadcasted_iota(jnp.int32, (BNJ, N), 1)
    onehot = (cand_ref[...] == col).astype(jnp.bfloat16)
    concat_ref[:, 0:HID] = jnp.dot(onehot, _bf(h2_ref[...]),
                                   preferred_element_type=jnp.float32)
    hp = hp_ref[...]
    for b in range(B):
        concat_ref[pl.ds(b * N_J, N_J), HID:2 * HID] = jnp.broadcast_to(
            hp[b:b + 1, :], (N_J, HID))


def kernel(adj, features, candidate, graph_pool,
           w1_0, b1_0, w2_0, b2_0, mlp_bn_g_0, mlp_bn_b_0, bn_g_0, bn_b_0,
           w1_1, b1_1, w2_1, b2_1, mlp_bn_g_1, mlp_bn_b_1, bn_g_1, bn_b_1):
    vec = lambda v: v.reshape(1, HID)
    full = lambda arr: pl.BlockSpec(arr.shape, lambda i: (0,) * arr.ndim)
    small = (_bf(graph_pool),
             _bf(w1_0), vec(b1_0), _bf(w2_0), vec(b2_0),
             vec(mlp_bn_g_0), vec(mlp_bn_b_0), vec(bn_g_0), vec(bn_b_0),
             _bf(w1_1), vec(b1_1), _bf(w2_1), vec(b2_1),
             vec(mlp_bn_g_1), vec(mlp_bn_b_1), vec(bn_g_1), vec(bn_b_1))
    h2, pooled = pl.pallas_call(
        _body,
        grid=(K + PC + 4,),
        in_specs=[
            pl.BlockSpec((BLK, N), lambda i: (jnp.minimum(i, K - 1), 0)),
            full(features),
        ] + [full(a) for a in small],
        out_specs=(
            pl.BlockSpec((N, HID), lambda i: (0, 0)),
            pl.BlockSpec((B, HID), lambda i: (0, 0)),
        ),
        out_shape=(
            jax.ShapeDtypeStruct((N, HID), jnp.float32),
            jax.ShapeDtypeStruct((B, HID), jnp.float32),
        ),
        scratch_shapes=[
            pltpu.VMEM((N, N), jnp.bfloat16),
            pltpu.VMEM((N, HID), jnp.float32),
            pltpu.VMEM((N, HID), jnp.bfloat16),
        ],
        compiler_params=pltpu.CompilerParams(
            vmem_limit_bytes=128 * 1024 * 1024,
        ),
    )(adj, _bf(features), *small)

    cand2 = (candidate.astype(jnp.int32)
             + jnp.arange(B, dtype=jnp.int32)[:, None] * NPG).reshape(BNJ, 1)
    concat2 = pl.pallas_call(
        _gather_body,
        out_shape=jax.ShapeDtypeStruct((BNJ, 2 * HID), jnp.float32),
    )(cand2, h2, pooled)
    return (concat2.reshape(B, N_J, 2 * HID), pooled)
